# Initial kernel scaffold; baseline (speedup 1.0000x reference)
import jax
import jax.numpy as jnp
from jax.experimental import pallas as pl


def kernel(indices, nids, data, weights):
    raise NotImplementedError("write your pallas kernel here")


# speedup vs baseline: 2.8792x; 2.8792x over previous
import jax
import jax.numpy as jnp
import numpy as np
from jax import lax
from jax.experimental import pallas as pl
from jax.experimental.pallas import tpu as pltpu
from jax.experimental.pallas import tpu_sc as plsc

D = 28
L = 16
NC, NS = 2, 16
NW = NC * NS
B = 128


def _worker_id():
    return lax.axis_index("s") * NC + lax.axis_index("c")


def _gather_start(src_hbm, idx_ref, dst, sem):
    return pltpu.async_copy(src_hbm.at[idx_ref], dst, sem)


def _body(idx_hbm, nids_hbm, data_hbm, w2_hbm, out_hbm,
          idx_v, cid_v, cix_v, w_v, rows_v, out_v, sem_n, sem_d):
    n = idx_hbm.shape[0]
    qpw = n // NW
    nblk = qpw // B
    wstart = _worker_id() * qpw
    nch = (B * 8) // 128
    io = lax.iota(jnp.int32, L)
    pat_q = lax.shift_right_logical(io, 3)
    pat_j = lax.bitwise_and(io, 7)

    def step(g, carry):
        qbase = wstart + g * B
        pltpu.sync_copy(idx_hbm.at[pl.ds(qbase, B)], idx_v)
        pltpu.sync_copy(w2_hbm.at[pl.ds(qbase // 2, B // 2)], w_v)
        _gather_start(nids_hbm, idx_v, cid_v, sem_n).wait()
        for k in range(B * 8 // L):
            v = plsc.load_gather(cid_v, [pat_q + 2 * k, pat_j])
            cix_v[k // 8, pl.ds((k % 8) * L, L)] = v
        gathers = [
            _gather_start(data_hbm, cix_v.at[c],
                          rows_v.at[pl.ds(c * 128, 128), :], sem_d)
            for c in range(nch)
        ]
        for gth in gathers:
            gth.wait()

        def qstep(p, c):
            w_pair = w_v[p, :]
            for h in range(2):
                q = 2 * p + h
                acc0 = jnp.zeros((L,), jnp.float32)
                acc1 = jnp.zeros((L,), jnp.float32)
                for j in range(8):
                    wj = w_pair.at[jnp.full((L,), 8 * h + j, jnp.int32)].get(
                        mode="promise_in_bounds")
                    r0 = rows_v[8 * q + j, pl.ds(0, L)]
                    r1 = rows_v[8 * q + j, pl.ds(D - L, L)]
                    acc0 = acc0 + wj * r0
                    acc1 = acc1 + wj * r1
                out_v[q, pl.ds(0, L)] = acc0
                out_v[q, pl.ds(D - L, L)] = acc1
            return c

        lax.fori_loop(0, B // 2, qstep, 0)
        pltpu.sync_copy(out_v, out_hbm.at[pl.ds(qbase, B)])
        return carry

    lax.fori_loop(0, nblk, step, 0)


def kernel(indices, nids, data, weights):
    n = indices.shape[0]
    w2 = weights.reshape(n // 2, 16)
    data32 = jnp.pad(data, ((0, 0), (0, 32 - D)))
    mesh = plsc.VectorSubcoreMesh(core_axis_name="c", subcore_axis_name="s",
                                  num_cores=NC, num_subcores=NS)
    f = pl.kernel(
        _body,
        out_type=jax.ShapeDtypeStruct((n, D), jnp.float32),
        mesh=mesh,
        compiler_params=pltpu.CompilerParams(use_tc_tiling_on_sc=False,
                                             needs_layout_passes=False),
        scratch_types=[
            pltpu.VMEM((B,), jnp.int32),
            pltpu.VMEM((B, 8), jnp.int32),
            pltpu.VMEM((B * 8 // 128, 128), jnp.int32),
            pltpu.VMEM((B // 2, 16), jnp.float32),
            pltpu.VMEM((B * 8, 32), jnp.float32),
            pltpu.VMEM((B, D), jnp.float32),
            pltpu.SemaphoreType.DMA,
            pltpu.SemaphoreType.DMA,
        ],
    )
    return f(indices, nids, data32, w2)
